# baseline (speedup 1.0000x reference)
import functools

import jax
import jax.numpy as jnp
from jax import lax
from jax.experimental import pallas as pl
from jax.experimental.pallas import tpu as pltpu
from jax.experimental.pallas import tpu_sc as plsc

B = 16384
D = 32
NC = 2
NS = 16
NW = NC * NS
RPW = B // NW
CH = 32
NCH = RPW // CH
EROW = 128
EMAJ = 250000


def _rsqrt(x):
    i = lax.bitcast_convert_type(x, jnp.int32)
    y = lax.bitcast_convert_type(jnp.int32(0x5F3759DF) - (i >> 1),
                                 jnp.float32)
    for _ in range(3):
        y = y * (1.5 - 0.5 * x * y * y)
    return y


@functools.partial(
    pl.kernel,
    mesh=plsc.VectorSubcoreMesh(core_axis_name="c", subcore_axis_name="s"),
    compiler_params=pltpu.CompilerParams(
        needs_layout_passes=False, use_tc_tiling_on_sc=False),
    out_type=(
        jax.ShapeDtypeStruct((B, D), jnp.float32),
        jax.ShapeDtypeStruct((B, D), jnp.float32),
        jax.ShapeDtypeStruct((B, D), jnp.float32),
    ),
    scratch_types=[
        pltpu.VMEM((NCH, CH), jnp.int32),
        pltpu.VMEM((4, 128), jnp.int32),
        pltpu.VMEM((NCH, CH), jnp.int32),
        pltpu.VMEM((NCH, CH), jnp.int32),
        pltpu.VMEM((NCH, CH), jnp.int32),
        pltpu.VMEM((2, CH, EROW), jnp.float32),
        pltpu.VMEM((2, CH, EROW), jnp.float32),
        pltpu.VMEM((2, CH, EROW), jnp.float32),
        pltpu.VMEM((2, CH, EROW), jnp.float32),
        pltpu.VMEM((RPW, D), jnp.float32),
        pltpu.VMEM((RPW, D), jnp.float32),
        pltpu.VMEM((RPW, D), jnp.float32),
        pltpu.VMEM((RPW, D), jnp.float32),
        pltpu.SemaphoreType.DMA,
        pltpu.SemaphoreType.DMA,
        pltpu.SemaphoreType.DMA,
    ],
)
def _transd_sc(h3, r3, t3, e2, p2, rE, rEP, hout, reout, tout,
               hv, rv, tv, hm, tm,
               heb, hpb, teb, tpb, rp, reb, ho, to,
               sem_r, sem_e0, sem_e1):
    wid = lax.axis_index("s") * NC + lax.axis_index("c")
    pltpu.sync_copy(h3.at[wid], hv)
    pltpu.sync_copy(r3.at[wid], rv)
    pltpu.sync_copy(t3.at[wid], tv)

    rel = []
    for k in range(4):
        sl = pl.ds(k * 128, 128)
        rel.append(pltpu.async_copy(rEP.at[rv.at[k]], rp.at[sl], sem_r))
        rel.append(pltpu.async_copy(rE.at[rv.at[k]], reb.at[sl], sem_r))

    for c in range(NCH):
        for s16 in range(CH // 16):
            sl = pl.ds(s16 * 16, 16)
            hm[c, sl] = hv[c, sl] >> 2
            tm[c, sl] = tv[c, sl] >> 2

    sems = [sem_e0, sem_e1]

    def fire(g, buf):
        pltpu.async_copy(e2.at[hm.at[g]], heb.at[buf], sems[buf])
        pltpu.async_copy(p2.at[hm.at[g]], hpb.at[buf], sems[buf])
        pltpu.async_copy(e2.at[tm.at[g]], teb.at[buf], sems[buf])
        pltpu.async_copy(p2.at[tm.at[g]], tpb.at[buf], sems[buf])

    def drain(g, buf):
        pltpu.make_async_copy(e2.at[hm.at[g]], heb.at[buf], sems[buf]).wait()
        pltpu.make_async_copy(p2.at[hm.at[g]], hpb.at[buf], sems[buf]).wait()
        pltpu.make_async_copy(e2.at[tm.at[g]], teb.at[buf], sems[buf]).wait()
        pltpu.make_async_copy(p2.at[tm.at[g]], tpb.at[buf], sems[buf]).wait()

    fire(0, 0)
    for c in rel:
        c.wait()

    lanes = lax.broadcasted_iota(jnp.int32, (16,), 0)
    one = jnp.float32(1.0)
    zero = jnp.zeros((16,), jnp.float32)

    def group16(g, buf, grp):
        bsl = pl.ds(grp * 16, 16)
        rows = grp * 16 + lanes
        r0 = g * CH + grp * 16 + lanes
        hcol = (hv[g, bsl] & 3) * D
        tcol = (tv[g, bsl] & 3) * D
        s_hp = s_he = d_h = s_tp = s_te = d_t = s_rp = s_re = zero
        for j in range(D):
            jv = jnp.full((16,), j, jnp.int32)
            hpj = plsc.load_gather(hpb.at[buf], [rows, hcol + j])
            hej = plsc.load_gather(heb.at[buf], [rows, hcol + j])
            tpj = plsc.load_gather(tpb.at[buf], [rows, tcol + j])
            tej = plsc.load_gather(teb.at[buf], [rows, tcol + j])
            rpj = plsc.load_gather(rp, [r0, jv])
            rej = plsc.load_gather(reb, [r0, jv])
            s_hp = s_hp + hpj * hpj
            s_he = s_he + hej * hej
            d_h = d_h + hpj * hej
            s_tp = s_tp + tpj * tpj
            s_te = s_te + tej * tej
            d_t = d_t + tpj * tej
            s_rp = s_rp + rpj * rpj
            s_re = s_re + rej * rej
        c_hp = jnp.minimum(one, _rsqrt(s_hp))
        c_he = jnp.minimum(one, _rsqrt(s_he))
        c_tp = jnp.minimum(one, _rsqrt(s_tp))
        c_te = jnp.minimum(one, _rsqrt(s_te))
        c_rp = jnp.minimum(one, _rsqrt(s_rp))
        c_re = jnp.minimum(one, _rsqrt(s_re))
        f_h = c_rp * c_hp * c_he * d_h
        f_t = c_rp * c_tp * c_te * d_t
        for j in range(D):
            jv = jnp.full((16,), j, jnp.int32)
            rpj = plsc.load_gather(rp, [r0, jv])
            hej = plsc.load_gather(heb.at[buf], [rows, hcol + j])
            tej = plsc.load_gather(teb.at[buf], [rows, tcol + j])
            rej = plsc.load_gather(reb, [r0, jv])
            plsc.store_scatter(ho, [r0, jv], f_h * rpj + c_he * hej)
            plsc.store_scatter(to, [r0, jv], f_t * rpj + c_te * tej)
            plsc.store_scatter(reb, [r0, jv], c_re * rej)

    def pair(outer, carry):
        for par in range(2):
            g = outer * 2 + par

            @pl.when(g + 1 < NCH)
            def _():
                fire(g + 1, 1 - par)

            drain(g, par)
            for grp in range(CH // 16):
                group16(g, par, grp)
        return carry

    lax.fori_loop(0, NCH // 2, pair, 0)

    out_sl = pl.ds(wid * RPW, RPW)
    pltpu.sync_copy(ho, hout.at[out_sl])
    pltpu.sync_copy(reb, reout.at[out_sl])
    pltpu.sync_copy(to, tout.at[out_sl])


def kernel(h, r, t, entityEmb, relationEmb, entityEmbP, relationEmbP):
    h3 = h.astype(jnp.int32).reshape(NW, NCH, CH)
    r3 = r.astype(jnp.int32).reshape(NW, 4, 128)
    t3 = t.astype(jnp.int32).reshape(NW, NCH, CH)
    e2 = entityEmb.reshape(EMAJ, EROW)
    p2 = entityEmbP.reshape(EMAJ, EROW)
    hout, reb, tout = _transd_sc(h3, r3, t3, e2, p2,
                                 relationEmb, relationEmbP)
    return (hout, reb, tout)
